# baseline (speedup 1.0000x reference)
import functools

import jax
import jax.numpy as jnp
from jax import lax
from jax.experimental import pallas as pl
from jax.experimental.pallas import tpu as pltpu
from jax.experimental.pallas import tpu_sc as plsc

B = 16384
D = 64
NC = 2
NS = 16
NW = NC * NS
BPW = B // NW
CH = 128
NCH = BPW // CH


def _body(words_hbm, ctx_hbm, w_hbm, c_hbm, out_hbm,
          widx_v, cidx_v, wrows_v, crows_v, sem_w, sem_c):
    wid = lax.axis_index("s") * NC + lax.axis_index("c")
    base = wid * BPW

    pltpu.sync_copy(words_hbm.at[wid], widx_v)
    pltpu.sync_copy(ctx_hbm.at[wid], cidx_v)

    w_copies = [
        pltpu.async_copy(w_hbm.at[widx_v.at[j]],
                         wrows_v.at[pl.ds(j * CH, CH)], sem_w)
        for j in range(NCH)
    ]
    c_copies = [
        pltpu.async_copy(c_hbm.at[cidx_v.at[j]],
                         crows_v.at[pl.ds(j * CH, CH)], sem_c)
        for j in range(NCH)
    ]
    for cp in w_copies:
        cp.wait()
    pltpu.sync_copy(wrows_v, out_hbm.at[0, pl.ds(base, BPW)])
    for cp in c_copies:
        cp.wait()
    pltpu.sync_copy(crows_v, out_hbm.at[1, pl.ds(base, BPW)])


@jax.jit
def _lookup(words, contexts, w_table, c_table):
    mesh = plsc.VectorSubcoreMesh(core_axis_name="c", subcore_axis_name="s")
    run = functools.partial(
        pl.kernel,
        mesh=mesh,
        out_type=jax.ShapeDtypeStruct((2, B, D), jnp.float32),
        scratch_types=[
            pltpu.VMEM((NCH, CH), jnp.int32),
            pltpu.VMEM((NCH, CH), jnp.int32),
            pltpu.VMEM((BPW, D), jnp.float32),
            pltpu.VMEM((BPW, D), jnp.float32),
            pltpu.SemaphoreType.DMA,
            pltpu.SemaphoreType.DMA,
        ],
        compiler_params=pltpu.CompilerParams(use_tc_tiling_on_sc=False),
    )(_body)
    return run(words, contexts, w_table, c_table)


def kernel(words, contexts, w_table, c_table):
    words = words.astype(jnp.int32).reshape(NW, NCH, CH)
    contexts = contexts.astype(jnp.int32).reshape(NW, NCH, CH)
    return _lookup(words, contexts, w_table, c_table)
